# Initial kernel scaffold; baseline (speedup 1.0000x reference)
import jax
import jax.numpy as jnp
from jax.experimental import pallas as pl


def kernel(x, k_edge_list, W, a):
    raise NotImplementedError("write your pallas kernel here")


# speedup vs baseline: 2.4044x; 2.4044x over previous
import functools

import jax
import jax.numpy as jnp
from jax import lax
from jax.experimental import pallas as pl
from jax.experimental.pallas import tpu as pltpu
from jax.experimental.pallas import tpu_sc as plsc

_N = 10000
_IN = 128
_OUT = 128
_HOP = 3
_E = 320000
_ALPHA = 0.2
_NC = 2
_NS = 16
_NW = _NC * _NS
_F = _OUT // _NW
_CH = 6400
_COEF = (0.5, 0.25, 0.125)


def _prep_body(x_ref, w_ref, a_ref, ht_ref, s1_ref, s2_ref):
    xb = x_ref[...]
    for i in range(_HOP):
        wi = w_ref[i * _IN:(i + 1) * _IN, :]
        ht = lax.dot_general(wi, xb, (((0,), (1,)), ((), ())),
                             preferred_element_type=jnp.float32)
        ht_ref[i] = ht
        a1 = a_ref[:, 2 * i * _OUT:2 * i * _OUT + _OUT]
        a2 = a_ref[:, 2 * i * _OUT + _OUT:2 * i * _OUT + 2 * _OUT]
        s1_ref[i] = lax.dot_general(a1, ht, (((1,), (0,)), ((), ())),
                                    preferred_element_type=jnp.float32)
        s2_ref[i] = lax.dot_general(a2, ht, (((1,), (0,)), ((), ())),
                                    preferred_element_type=jnp.float32)


def _prep(x, W, a):
    return pl.pallas_call(
        _prep_body,
        out_shape=[
            jax.ShapeDtypeStruct((_HOP, _OUT, _N), jnp.float32),
            jax.ShapeDtypeStruct((_HOP, 1, _N), jnp.float32),
            jax.ShapeDtypeStruct((_HOP, 1, _N), jnp.float32),
        ],
    )(x, W, a)


def _sc_edges(ht_r, s1, s2, edges, zeros):
    mesh = plsc.VectorSubcoreMesh(core_axis_name="c", subcore_axis_name="s")

    @functools.partial(
        pl.kernel,
        out_type=[
            jax.ShapeDtypeStruct((_HOP * _NW * _F * _N,), jnp.float32),
            jax.ShapeDtypeStruct((_HOP * _N,), jnp.float32),
        ],
        mesh=mesh,
        compiler_params=pltpu.CompilerParams(needs_layout_passes=False),
        scratch_types=[
            pltpu.VMEM((_F * _N,), jnp.float32),
            pltpu.VMEM((_F * _N,), jnp.float32),
            pltpu.VMEM((_N,), jnp.float32),
            pltpu.VMEM((_N,), jnp.float32),
            pltpu.VMEM((_N,), jnp.float32),
            pltpu.VMEM((_CH,), jnp.int32),
            pltpu.VMEM((_CH,), jnp.int32),
        ],
    )
    def k(ht_hbm, s1_hbm, s2_hbm, e_hbm, z_hbm, acc_hbm, den_hbm,
          h_loc, acc_loc, s1_loc, s2_loc, den_loc, srcb, dstb):
        wid = lax.axis_index("s") * _NC + lax.axis_index("c")
        for i in range(_HOP):
            pltpu.sync_copy(ht_hbm.at[pl.ds((i * _NW + wid) * _F * _N, _F * _N)],
                            h_loc)
            pltpu.sync_copy(s1_hbm.at[pl.ds(i * _N, _N)], s1_loc)
            pltpu.sync_copy(s2_hbm.at[pl.ds(i * _N, _N)], s2_loc)
            pltpu.sync_copy(z_hbm, acc_loc)
            pltpu.sync_copy(z_hbm.at[pl.ds(0, _N)], den_loc)

            def chunk_body(c, carry, i=i):
                pltpu.sync_copy(
                    e_hbm.at[pl.ds(2 * i * _E + c * _CH, _CH)], srcb)
                pltpu.sync_copy(
                    e_hbm.at[pl.ds((2 * i + 1) * _E + c * _CH, _CH)], dstb)

                def group_body(g, carry2):
                    b = g * 16
                    src_v = srcb[pl.ds(b, 16)]
                    dst_v = dstb[pl.ds(b, 16)]
                    s1v = plsc.load_gather(s1_loc, [src_v])
                    s2v = plsc.load_gather(s2_loc, [dst_v])
                    sc = s1v + s2v
                    lr = jnp.maximum(sc, _ALPHA * sc)
                    w = jnp.exp(-lr)
                    plsc.addupdate_scatter(den_loc, [src_v], w)
                    gi = dst_v
                    si = src_v
                    for f in range(_F):
                        vals = plsc.load_gather(h_loc, [gi])
                        plsc.addupdate_scatter(acc_loc, [si], vals * w)
                        if f < _F - 1:
                            gi = gi + _N
                            si = si + _N
                    return carry2

                return lax.fori_loop(0, _CH // 16, group_body, carry)

            lax.fori_loop(0, _E // _CH, chunk_body, 0)
            pltpu.sync_copy(
                acc_loc,
                acc_hbm.at[pl.ds((i * _NW + wid) * _F * _N, _F * _N)])

            @pl.when(wid == 0)
            def _(i=i):
                pltpu.sync_copy(den_loc, den_hbm.at[pl.ds(i * _N, _N)])

    return k(ht_r, s1, s2, edges, zeros)


def _finish_body(num_ref, den_ref, out_ref):
    acc = jnp.zeros((_OUT, _N), jnp.float32)
    for i in range(_HOP):
        acc = acc + _COEF[i] * num_ref[i] / den_ref[i]
    out_ref[...] = jnp.where(acc > 0, acc, jnp.exp(acc) - 1.0).T


def _finish(num, den):
    return pl.pallas_call(
        _finish_body,
        out_shape=jax.ShapeDtypeStruct((_N, _OUT), jnp.float32),
    )(num, den)


def kernel(x, k_edge_list, W, a):
    ht, s1, s2 = _prep(x, W, a)
    ht_r = ht.reshape(_HOP * _NW * _F * _N)
    zeros = jnp.zeros((_F * _N,), jnp.float32)
    acc, den = _sc_edges(ht_r, s1.reshape(_HOP * _N), s2.reshape(_HOP * _N),
                         k_edge_list.reshape(_HOP * 2 * _E), zeros)
    num = acc.reshape(_HOP, _OUT, _N)
    return _finish(num, den.reshape(_HOP, 1, _N))


# speedup vs baseline: 2.6434x; 1.0994x over previous
import functools

import jax
import jax.numpy as jnp
from jax import lax
from jax.experimental import pallas as pl
from jax.experimental.pallas import tpu as pltpu
from jax.experimental.pallas import tpu_sc as plsc

_N = 10000
_IN = 128
_OUT = 128
_HOP = 3
_E = 320000
_ALPHA = 0.2
_NC = 2
_NS = 16
_NW = _NC * _NS
_F = _OUT // _NW
_CH = 3200
_NCHUNK = _E // _CH
_GPI = 4
_COEF = (0.5, 0.25, 0.125)


def _prep_body(x_ref, w_ref, a_ref, ht_ref, s1_ref, s2_ref):
    xb = x_ref[...]
    for i in range(_HOP):
        wi = w_ref[i * _IN:(i + 1) * _IN, :]
        ht = lax.dot_general(wi, xb, (((0,), (1,)), ((), ())),
                             preferred_element_type=jnp.float32)
        ht_ref[i] = ht
        a1 = a_ref[:, 2 * i * _OUT:2 * i * _OUT + _OUT]
        a2 = a_ref[:, 2 * i * _OUT + _OUT:2 * i * _OUT + 2 * _OUT]
        s1_ref[i] = lax.dot_general(a1, ht, (((1,), (0,)), ((), ())),
                                    preferred_element_type=jnp.float32)
        s2_ref[i] = lax.dot_general(a2, ht, (((1,), (0,)), ((), ())),
                                    preferred_element_type=jnp.float32)


def _prep(x, W, a):
    return pl.pallas_call(
        _prep_body,
        out_shape=[
            jax.ShapeDtypeStruct((_HOP, _OUT, _N), jnp.float32),
            jax.ShapeDtypeStruct((_HOP, 1, _N), jnp.float32),
            jax.ShapeDtypeStruct((_HOP, 1, _N), jnp.float32),
        ],
    )(x, W, a)


def _sc_edges(ht_r, s1, s2, edges, zeros):
    mesh = plsc.VectorSubcoreMesh(core_axis_name="c", subcore_axis_name="s")

    @functools.partial(
        pl.kernel,
        out_type=[
            jax.ShapeDtypeStruct((_HOP * _NW * _F * _N,), jnp.float32),
            jax.ShapeDtypeStruct((_HOP * _N,), jnp.float32),
        ],
        mesh=mesh,
        compiler_params=pltpu.CompilerParams(needs_layout_passes=False),
        scratch_types=[
            pltpu.VMEM((_F * _N,), jnp.float32),
            pltpu.VMEM((_F * _N,), jnp.float32),
            pltpu.VMEM((_N,), jnp.float32),
            pltpu.VMEM((_N,), jnp.float32),
            pltpu.VMEM((_N,), jnp.float32),
            pltpu.VMEM((_CH,), jnp.int32),
            pltpu.VMEM((_CH,), jnp.int32),
            pltpu.VMEM((_CH,), jnp.int32),
            pltpu.VMEM((_CH,), jnp.int32),
            pltpu.SemaphoreType.DMA,
            pltpu.SemaphoreType.DMA,
            pltpu.SemaphoreType.DMA,
            pltpu.SemaphoreType.DMA,
        ],
    )
    def k(ht_hbm, s1_hbm, s2_hbm, e_hbm, z_hbm, acc_hbm, den_hbm,
          h_loc, acc_loc, s1_loc, s2_loc, den_loc,
          srcA, dstA, srcB, dstB, semSA, semDA, semSB, semDB):
        wid = lax.axis_index("s") * _NC + lax.axis_index("c")

        def issue(i, c, sbuf, dbuf, ssem, dsem):
            pltpu.async_copy(
                e_hbm.at[pl.ds(2 * i * _E + c * _CH, _CH)], sbuf, ssem)
            pltpu.async_copy(
                e_hbm.at[pl.ds((2 * i + 1) * _E + c * _CH, _CH)], dbuf, dsem)

        def drain(sbuf, dbuf, ssem, dsem):
            pltpu.make_async_copy(e_hbm.at[pl.ds(0, _CH)], sbuf, ssem).wait()
            pltpu.make_async_copy(e_hbm.at[pl.ds(0, _CH)], dbuf, dsem).wait()

        def group(sb, db, base):
            src_v = sb[pl.ds(base, 16)]
            dst_v = db[pl.ds(base, 16)]
            s1v = plsc.load_gather(s1_loc, [src_v])
            s2v = plsc.load_gather(s2_loc, [dst_v])
            sc = s1v + s2v
            w = jnp.exp(-jnp.maximum(sc, _ALPHA * sc))
            plsc.addupdate_scatter(den_loc, [src_v], w)
            gi = dst_v
            si = src_v
            for f in range(_F):
                vals = plsc.load_gather(h_loc, [gi])
                plsc.addupdate_scatter(acc_loc, [si], vals * w)
                if f < _F - 1:
                    gi = gi + _N
                    si = si + _N

        def process(sb, db):
            def group_body(g, carry):
                for u in range(_GPI):
                    group(sb, db, g * (16 * _GPI) + u * 16)
                return carry
            lax.fori_loop(0, _CH // (16 * _GPI), group_body, 0)

        for i in range(_HOP):
            pltpu.sync_copy(ht_hbm.at[pl.ds((i * _NW + wid) * _F * _N, _F * _N)],
                            h_loc)
            pltpu.sync_copy(s1_hbm.at[pl.ds(i * _N, _N)], s1_loc)
            pltpu.sync_copy(s2_hbm.at[pl.ds(i * _N, _N)], s2_loc)
            pltpu.sync_copy(z_hbm, acc_loc)
            pltpu.sync_copy(z_hbm.at[pl.ds(0, _N)], den_loc)

            issue(i, 0, srcA, dstA, semSA, semDA)

            def chunk_pair(c2, carry, i=i):
                c1 = 2 * c2 + 1
                cn = jnp.minimum(2 * c2 + 2, _NCHUNK - 1)
                drain(srcA, dstA, semSA, semDA)
                issue(i, c1, srcB, dstB, semSB, semDB)
                process(srcA, dstA)
                drain(srcB, dstB, semSB, semDB)
                issue(i, cn, srcA, dstA, semSA, semDA)
                process(srcB, dstB)
                return carry

            lax.fori_loop(0, _NCHUNK // 2, chunk_pair, 0)
            drain(srcA, dstA, semSA, semDA)
            pltpu.sync_copy(
                acc_loc,
                acc_hbm.at[pl.ds((i * _NW + wid) * _F * _N, _F * _N)])

            @pl.when(wid == 0)
            def _(i=i):
                pltpu.sync_copy(den_loc, den_hbm.at[pl.ds(i * _N, _N)])

    return k(ht_r, s1, s2, edges, zeros)


def _finish_body(num_ref, den_ref, out_ref):
    acc = jnp.zeros((_OUT, _N), jnp.float32)
    for i in range(_HOP):
        acc = acc + _COEF[i] * num_ref[i] / den_ref[i]
    out_ref[...] = jnp.where(acc > 0, acc, jnp.exp(acc) - 1.0).T


def _finish(num, den):
    return pl.pallas_call(
        _finish_body,
        out_shape=jax.ShapeDtypeStruct((_N, _OUT), jnp.float32),
    )(num, den)


def kernel(x, k_edge_list, W, a):
    ht, s1, s2 = _prep(x, W, a)
    ht_r = ht.reshape(_HOP * _NW * _F * _N)
    zeros = jnp.zeros((_F * _N,), jnp.float32)
    acc, den = _sc_edges(ht_r, s1.reshape(_HOP * _N), s2.reshape(_HOP * _N),
                         k_edge_list.reshape(_HOP * 2 * _E), zeros)
    num = acc.reshape(_HOP, _OUT, _N)
    return _finish(num, den.reshape(_HOP, 1, _N))


# speedup vs baseline: 7.1100x; 2.6897x over previous
import functools

import jax
import jax.numpy as jnp
from jax import lax
from jax.experimental import pallas as pl
from jax.experimental.pallas import tpu as pltpu
from jax.experimental.pallas import tpu_sc as plsc

_N = 10000
_IN = 128
_OUT = 128
_HOP = 3
_E = 320000
_ALPHA = 0.2
_NC = 2
_NS = 16
_NW = _NC * _NS
_F = _OUT // _NW
_CH = 3200
_NCHUNK = _E // _CH
_GPI = 8
_COEF = (0.5, 0.25, 0.125)


def _prep_body(x_ref, w_ref, a_ref, ht_ref, s1_ref, s2_ref):
    xb = x_ref[...]
    for i in range(_HOP):
        wi = w_ref[i * _IN:(i + 1) * _IN, :]
        ht = lax.dot_general(wi, xb, (((0,), (1,)), ((), ())),
                             preferred_element_type=jnp.float32)
        ht_ref[i] = ht
        a1 = a_ref[:, 2 * i * _OUT:2 * i * _OUT + _OUT]
        a2 = a_ref[:, 2 * i * _OUT + _OUT:2 * i * _OUT + 2 * _OUT]
        s1_ref[i] = lax.dot_general(a1, ht, (((1,), (0,)), ((), ())),
                                    preferred_element_type=jnp.float32)
        s2_ref[i] = lax.dot_general(a2, ht, (((1,), (0,)), ((), ())),
                                    preferred_element_type=jnp.float32)


def _prep(x, W, a):
    return pl.pallas_call(
        _prep_body,
        out_shape=[
            jax.ShapeDtypeStruct((_HOP, _OUT, _N), jnp.float32),
            jax.ShapeDtypeStruct((_HOP, 1, _N), jnp.float32),
            jax.ShapeDtypeStruct((_HOP, 1, _N), jnp.float32),
        ],
    )(x, W, a)


def _sc_edges(ht_r, s1, s2, edges, zeros):
    mesh = plsc.VectorSubcoreMesh(core_axis_name="c", subcore_axis_name="s")

    @functools.partial(
        pl.kernel,
        out_type=[
            jax.ShapeDtypeStruct((_HOP * _NW * _F * _N,), jnp.float32),
            jax.ShapeDtypeStruct((_HOP * _N,), jnp.float32),
        ],
        mesh=mesh,
        compiler_params=pltpu.CompilerParams(needs_layout_passes=False),
        scratch_types=[
            pltpu.VMEM((_F * _N,), jnp.float32),
            pltpu.VMEM((_F * _N,), jnp.float32),
            pltpu.VMEM((_N,), jnp.float32),
            pltpu.VMEM((_N,), jnp.float32),
            pltpu.VMEM((_N,), jnp.float32),
            pltpu.VMEM((_CH,), jnp.int32),
            pltpu.VMEM((_CH,), jnp.int32),
            pltpu.VMEM((_CH,), jnp.int32),
            pltpu.VMEM((_CH,), jnp.int32),
            pltpu.SemaphoreType.DMA,
            pltpu.SemaphoreType.DMA,
            pltpu.SemaphoreType.DMA,
            pltpu.SemaphoreType.DMA,
        ],
    )
    def k(ht_hbm, s1_hbm, s2_hbm, e_hbm, z_hbm, acc_hbm, den_hbm,
          h_loc, acc_loc, s1_loc, s2_loc, den_loc,
          srcA, dstA, srcB, dstB, semSA, semDA, semSB, semDB):
        wid = lax.axis_index("s") * _NC + lax.axis_index("c")

        def issue(i, c, sbuf, dbuf, ssem, dsem):
            pltpu.async_copy(
                e_hbm.at[pl.ds(2 * i * _E + c * _CH, _CH)], sbuf, ssem)
            pltpu.async_copy(
                e_hbm.at[pl.ds((2 * i + 1) * _E + c * _CH, _CH)], dbuf, dsem)

        def drain(sbuf, dbuf, ssem, dsem):
            pltpu.make_async_copy(e_hbm.at[pl.ds(0, _CH)], sbuf, ssem).wait()
            pltpu.make_async_copy(e_hbm.at[pl.ds(0, _CH)], dbuf, dsem).wait()

        def group(sb, db, base):
            src_v = sb[pl.ds(base, 16)]
            dst_v = db[pl.ds(base, 16)]
            s1v = plsc.load_gather(s1_loc, [src_v])
            s2v = plsc.load_gather(s2_loc, [dst_v])
            sc = s1v + s2v
            w = jnp.exp(-jnp.maximum(sc, _ALPHA * sc))
            plsc.addupdate_scatter(den_loc, [src_v], w)
            gi = dst_v
            si = src_v
            for f in range(_F):
                vals = plsc.load_gather(h_loc, [gi])
                plsc.addupdate_scatter(acc_loc, [si], vals * w)
                if f < _F - 1:
                    gi = gi + _N
                    si = si + _N

        def process(sb, db):
            @plsc.parallel_loop(0, _CH // 16, 1, unroll=_GPI)
            def _body(g):
                group(sb, db, g * 16)

        for i in range(_HOP):
            pltpu.sync_copy(ht_hbm.at[pl.ds((i * _NW + wid) * _F * _N, _F * _N)],
                            h_loc)
            pltpu.sync_copy(s1_hbm.at[pl.ds(i * _N, _N)], s1_loc)
            pltpu.sync_copy(s2_hbm.at[pl.ds(i * _N, _N)], s2_loc)
            pltpu.sync_copy(z_hbm, acc_loc)
            pltpu.sync_copy(z_hbm.at[pl.ds(0, _N)], den_loc)

            issue(i, 0, srcA, dstA, semSA, semDA)

            def chunk_pair(c2, carry, i=i):
                c1 = 2 * c2 + 1
                cn = jnp.minimum(2 * c2 + 2, _NCHUNK - 1)
                drain(srcA, dstA, semSA, semDA)
                issue(i, c1, srcB, dstB, semSB, semDB)
                process(srcA, dstA)
                drain(srcB, dstB, semSB, semDB)
                issue(i, cn, srcA, dstA, semSA, semDA)
                process(srcB, dstB)
                return carry

            lax.fori_loop(0, _NCHUNK // 2, chunk_pair, 0)
            drain(srcA, dstA, semSA, semDA)
            pltpu.sync_copy(
                acc_loc,
                acc_hbm.at[pl.ds((i * _NW + wid) * _F * _N, _F * _N)])

            @pl.when(wid == 0)
            def _(i=i):
                pltpu.sync_copy(den_loc, den_hbm.at[pl.ds(i * _N, _N)])

    return k(ht_r, s1, s2, edges, zeros)


def _finish_body(num_ref, den_ref, out_ref):
    acc = jnp.zeros((_OUT, _N), jnp.float32)
    for i in range(_HOP):
        acc = acc + _COEF[i] * num_ref[i] / den_ref[i]
    out_ref[...] = jnp.where(acc > 0, acc, jnp.exp(acc) - 1.0).T


def _finish(num, den):
    return pl.pallas_call(
        _finish_body,
        out_shape=jax.ShapeDtypeStruct((_N, _OUT), jnp.float32),
    )(num, den)


def kernel(x, k_edge_list, W, a):
    ht, s1, s2 = _prep(x, W, a)
    ht_r = ht.reshape(_HOP * _NW * _F * _N)
    zeros = jnp.zeros((_F * _N,), jnp.float32)
    acc, den = _sc_edges(ht_r, s1.reshape(_HOP * _N), s2.reshape(_HOP * _N),
                         k_edge_list.reshape(_HOP * 2 * _E), zeros)
    num = acc.reshape(_HOP, _OUT, _N)
    return _finish(num, den.reshape(_HOP, 1, _N))


# speedup vs baseline: 8.8742x; 1.2481x over previous
import functools

import jax
import jax.numpy as jnp
from jax import lax
from jax.experimental import pallas as pl
from jax.experimental.pallas import tpu as pltpu
from jax.experimental.pallas import tpu_sc as plsc

_N = 10000
_IN = 128
_OUT = 128
_HOP = 3
_E = 320000
_ALPHA = 0.2
_NC = 2
_NS = 16
_NW = _NC * _NS
_F = _OUT // _NW
_CH = 3200
_NCHUNK = _E // _CH
_GPI = 8
_COEF = (0.5, 0.25, 0.125)


def _prep_body(x_ref, w_ref, a_ref, ht_ref, s1_ref, s2_ref):
    xb = x_ref[...]
    for i in range(_HOP):
        wi = w_ref[i * _IN:(i + 1) * _IN, :]
        ht = lax.dot_general(wi, xb, (((0,), (1,)), ((), ())),
                             preferred_element_type=jnp.float32)
        ht_ref[i] = ht
        a1 = a_ref[:, 2 * i * _OUT:2 * i * _OUT + _OUT]
        a2 = a_ref[:, 2 * i * _OUT + _OUT:2 * i * _OUT + 2 * _OUT]
        s1_ref[i] = lax.dot_general(a1, ht, (((1,), (0,)), ((), ())),
                                    preferred_element_type=jnp.float32)
        s2_ref[i] = lax.dot_general(a2, ht, (((1,), (0,)), ((), ())),
                                    preferred_element_type=jnp.float32)


def _prep(x, W, a):
    return pl.pallas_call(
        _prep_body,
        out_shape=[
            jax.ShapeDtypeStruct((_HOP, _OUT, _N), jnp.float32),
            jax.ShapeDtypeStruct((_HOP, 1, _N), jnp.float32),
            jax.ShapeDtypeStruct((_HOP, 1, _N), jnp.float32),
        ],
    )(x, W, a)


def _sc_edges(ht_r, s1, s2, edges, zeros):
    mesh = plsc.VectorSubcoreMesh(core_axis_name="c", subcore_axis_name="s")

    @functools.partial(
        pl.kernel,
        out_type=[
            jax.ShapeDtypeStruct((_HOP * _NW * _F * _N,), jnp.float32),
            jax.ShapeDtypeStruct((_HOP * _N,), jnp.float32),
        ],
        mesh=mesh,
        compiler_params=pltpu.CompilerParams(needs_layout_passes=False),
        scratch_types=[
            pltpu.VMEM((_F * _N,), jnp.float32),
            pltpu.VMEM((_F * _N,), jnp.float32),
            pltpu.VMEM((_N,), jnp.float32),
            pltpu.VMEM((_N,), jnp.float32),
            pltpu.VMEM((_N,), jnp.float32),
            pltpu.VMEM((_CH,), jnp.int32),
            pltpu.VMEM((_CH,), jnp.int32),
            pltpu.VMEM((_CH,), jnp.int32),
            pltpu.VMEM((_CH,), jnp.int32),
            pltpu.SemaphoreType.DMA,
            pltpu.SemaphoreType.DMA,
            pltpu.SemaphoreType.DMA,
            pltpu.SemaphoreType.DMA,
        ],
    )
    def k(ht_hbm, s1_hbm, s2_hbm, e_hbm, z_hbm, acc_hbm, den_hbm,
          h_loc, acc_loc, s1_loc, s2_loc, den_loc,
          srcA, dstA, srcB, dstB, semSA, semDA, semSB, semDB):
        wid = lax.axis_index("s") * _NC + lax.axis_index("c")

        def issue(i, c, sbuf, dbuf, ssem, dsem):
            pltpu.async_copy(
                e_hbm.at[pl.ds(2 * i * _E + c * _CH, _CH)], sbuf, ssem)
            pltpu.async_copy(
                e_hbm.at[pl.ds((2 * i + 1) * _E + c * _CH, _CH)], dbuf, dsem)

        def drain(sbuf, dbuf, ssem, dsem):
            pltpu.make_async_copy(e_hbm.at[pl.ds(0, _CH)], sbuf, ssem).wait()
            pltpu.make_async_copy(e_hbm.at[pl.ds(0, _CH)], dbuf, dsem).wait()

        def group(sb, db, base):
            src_v = sb[pl.ds(base, 16)]
            dst_v = db[pl.ds(base, 16)]
            s1v = plsc.load_gather(s1_loc, [src_v])
            s2v = plsc.load_gather(s2_loc, [dst_v])
            sc = s1v + s2v
            w = jnp.exp(-jnp.maximum(sc, _ALPHA * sc))
            plsc.addupdate_scatter(den_loc, [src_v], w)
            for f in range(_F):
                vals = plsc.load_gather(h_loc.at[pl.ds(f * _N, _N)], [dst_v])
                plsc.addupdate_scatter(acc_loc.at[pl.ds(f * _N, _N)],
                                       [src_v], vals * w)

        def process(sb, db):
            @plsc.parallel_loop(0, _CH // 16, 1, unroll=_GPI)
            def _body(g):
                group(sb, db, g * 16)

        for i in range(_HOP):
            pltpu.sync_copy(ht_hbm.at[pl.ds((i * _NW + wid) * _F * _N, _F * _N)],
                            h_loc)
            pltpu.sync_copy(s1_hbm.at[pl.ds(i * _N, _N)], s1_loc)
            pltpu.sync_copy(s2_hbm.at[pl.ds(i * _N, _N)], s2_loc)
            pltpu.sync_copy(z_hbm, acc_loc)
            pltpu.sync_copy(z_hbm.at[pl.ds(0, _N)], den_loc)

            issue(i, 0, srcA, dstA, semSA, semDA)

            def chunk_pair(c2, carry, i=i):
                c1 = 2 * c2 + 1
                cn = jnp.minimum(2 * c2 + 2, _NCHUNK - 1)
                drain(srcA, dstA, semSA, semDA)
                issue(i, c1, srcB, dstB, semSB, semDB)
                process(srcA, dstA)
                drain(srcB, dstB, semSB, semDB)
                issue(i, cn, srcA, dstA, semSA, semDA)
                process(srcB, dstB)
                return carry

            lax.fori_loop(0, _NCHUNK // 2, chunk_pair, 0)
            drain(srcA, dstA, semSA, semDA)
            pltpu.sync_copy(
                acc_loc,
                acc_hbm.at[pl.ds((i * _NW + wid) * _F * _N, _F * _N)])

            @pl.when(wid == 0)
            def _(i=i):
                pltpu.sync_copy(den_loc, den_hbm.at[pl.ds(i * _N, _N)])

    return k(ht_r, s1, s2, edges, zeros)


def _finish_body(num_ref, den_ref, out_ref):
    acc = jnp.zeros((_OUT, _N), jnp.float32)
    for i in range(_HOP):
        acc = acc + _COEF[i] * num_ref[i] / den_ref[i]
    out_ref[...] = jnp.where(acc > 0, acc, jnp.exp(acc) - 1.0).T


def _finish(num, den):
    return pl.pallas_call(
        _finish_body,
        out_shape=jax.ShapeDtypeStruct((_N, _OUT), jnp.float32),
    )(num, den)


def kernel(x, k_edge_list, W, a):
    ht, s1, s2 = _prep(x, W, a)
    ht_r = ht.reshape(_HOP * _NW * _F * _N)
    zeros = jnp.zeros((_F * _N,), jnp.float32)
    acc, den = _sc_edges(ht_r, s1.reshape(_HOP * _N), s2.reshape(_HOP * _N),
                         k_edge_list.reshape(_HOP * 2 * _E), zeros)
    num = acc.reshape(_HOP, _OUT, _N)
    return _finish(num, den.reshape(_HOP, 1, _N))


# speedup vs baseline: 9.8244x; 1.1071x over previous
import functools

import jax
import jax.numpy as jnp
from jax import lax
from jax.experimental import pallas as pl
from jax.experimental.pallas import tpu as pltpu
from jax.experimental.pallas import tpu_sc as plsc

_N = 10000
_IN = 128
_OUT = 128
_HOP = 3
_E = 320000
_ALPHA = 0.2
_NC = 2
_NS = 16
_NW = _NC * _NS
_F = _OUT // _NW
_CH = 3200
_NCHUNK = _E // _CH
_GPI = 8
_COEF = (0.5, 0.25, 0.125)


def _prep_body(x_ref, w_ref, a_ref, hp_ref, s1_ref, s2_ref):
    xb = x_ref[...]
    _H = _OUT // 2
    for i in range(_HOP):
        wi_e = w_ref[i * _IN:(i + 1) * _IN, :_H]
        wi_o = w_ref[i * _IN:(i + 1) * _IN, _H:]
        hte = lax.dot_general(wi_e, xb, (((0,), (1,)), ((), ())),
                              preferred_element_type=jnp.float32)
        hto = lax.dot_general(wi_o, xb, (((0,), (1,)), ((), ())),
                              preferred_element_type=jnp.float32)
        hue = lax.bitcast_convert_type(
            hte.astype(jnp.bfloat16), jnp.uint16).astype(jnp.uint32)
        huo = lax.bitcast_convert_type(
            hto.astype(jnp.bfloat16), jnp.uint16).astype(jnp.uint32)
        hp_ref[i] = lax.bitcast_convert_type(hue | (huo << 16), jnp.int32)
        a1e = a_ref[:, 2 * i * _OUT:2 * i * _OUT + _H]
        a1o = a_ref[:, 2 * i * _OUT + _H:2 * i * _OUT + _OUT]
        a2e = a_ref[:, 2 * i * _OUT + _OUT:2 * i * _OUT + _OUT + _H]
        a2o = a_ref[:, 2 * i * _OUT + _OUT + _H:2 * (i + 1) * _OUT]
        s1_ref[i] = (
            lax.dot_general(a1e, hte, (((1,), (0,)), ((), ())),
                            preferred_element_type=jnp.float32)
            + lax.dot_general(a1o, hto, (((1,), (0,)), ((), ())),
                              preferred_element_type=jnp.float32))
        s2_ref[i] = (
            lax.dot_general(a2e, hte, (((1,), (0,)), ((), ())),
                            preferred_element_type=jnp.float32)
            + lax.dot_general(a2o, hto, (((1,), (0,)), ((), ())),
                              preferred_element_type=jnp.float32))


def _prep(x, W, a):
    return pl.pallas_call(
        _prep_body,
        out_shape=[
            jax.ShapeDtypeStruct((_HOP, _OUT // 2, _N), jnp.int32),
            jax.ShapeDtypeStruct((_HOP, 1, _N), jnp.float32),
            jax.ShapeDtypeStruct((_HOP, 1, _N), jnp.float32),
        ],
    )(x, W, a)


def _sc_edges(ht_r, s1, s2, edges, zeros):
    mesh = plsc.VectorSubcoreMesh(core_axis_name="c", subcore_axis_name="s")

    @functools.partial(
        pl.kernel,
        out_type=[
            jax.ShapeDtypeStruct((_HOP * _NW * _F * _N,), jnp.float32),
            jax.ShapeDtypeStruct((_HOP * _N,), jnp.float32),
        ],
        mesh=mesh,
        compiler_params=pltpu.CompilerParams(needs_layout_passes=False),
        scratch_types=[
            pltpu.VMEM(((_F // 2) * _N,), jnp.int32),
            pltpu.VMEM((_F * _N,), jnp.float32),
            pltpu.VMEM((_N,), jnp.float32),
            pltpu.VMEM((_N,), jnp.float32),
            pltpu.VMEM((_N,), jnp.float32),
            pltpu.VMEM((_CH,), jnp.int32),
            pltpu.VMEM((_CH,), jnp.int32),
            pltpu.VMEM((_CH,), jnp.int32),
            pltpu.VMEM((_CH,), jnp.int32),
            pltpu.SemaphoreType.DMA,
            pltpu.SemaphoreType.DMA,
            pltpu.SemaphoreType.DMA,
            pltpu.SemaphoreType.DMA,
        ],
    )
    def k(ht_hbm, s1_hbm, s2_hbm, e_hbm, z_hbm, acc_hbm, den_hbm,
          h_loc, acc_loc, s1_loc, s2_loc, den_loc,
          srcA, dstA, srcB, dstB, semSA, semDA, semSB, semDB):
        wid = lax.axis_index("s") * _NC + lax.axis_index("c")

        def issue(i, c, sbuf, dbuf, ssem, dsem):
            pltpu.async_copy(
                e_hbm.at[pl.ds(2 * i * _E + c * _CH, _CH)], sbuf, ssem)
            pltpu.async_copy(
                e_hbm.at[pl.ds((2 * i + 1) * _E + c * _CH, _CH)], dbuf, dsem)

        def drain(sbuf, dbuf, ssem, dsem):
            pltpu.make_async_copy(e_hbm.at[pl.ds(0, _CH)], sbuf, ssem).wait()
            pltpu.make_async_copy(e_hbm.at[pl.ds(0, _CH)], dbuf, dsem).wait()

        def group(sb, db, base):
            src_v = sb[pl.ds(base, 16)]
            dst_v = db[pl.ds(base, 16)]
            s1v = plsc.load_gather(s1_loc, [src_v])
            s2v = plsc.load_gather(s2_loc, [dst_v])
            sc = s1v + s2v
            w = jnp.exp(-jnp.maximum(sc, _ALPHA * sc))
            plsc.addupdate_scatter(den_loc, [src_v], w)
            for p in range(_F // 2):
                pw = plsc.load_gather(h_loc.at[pl.ds(p * _N, _N)], [dst_v])
                va, vb = plsc.unpack(plsc.bitcast(pw, jnp.bfloat16),
                                     format=plsc.PackFormat.INTERLEAVED,
                                     preferred_element_type=jnp.float32)
                plsc.addupdate_scatter(acc_loc.at[pl.ds(2 * p * _N, _N)],
                                       [src_v], va * w)
                plsc.addupdate_scatter(acc_loc.at[pl.ds((2 * p + 1) * _N, _N)],
                                       [src_v], vb * w)

        def process(sb, db):
            @plsc.parallel_loop(0, _CH // 16, 1, unroll=_GPI)
            def _body(g):
                group(sb, db, g * 16)

        _FP = _F // 2
        for i in range(_HOP):
            pltpu.sync_copy(
                ht_hbm.at[pl.ds((i * _NW + wid) * _FP * _N, _FP * _N)], h_loc)
            pltpu.sync_copy(s1_hbm.at[pl.ds(i * _N, _N)], s1_loc)
            pltpu.sync_copy(s2_hbm.at[pl.ds(i * _N, _N)], s2_loc)
            pltpu.sync_copy(z_hbm, acc_loc)
            pltpu.sync_copy(z_hbm.at[pl.ds(0, _N)], den_loc)

            issue(i, 0, srcA, dstA, semSA, semDA)

            def chunk_pair(c2, carry, i=i):
                c1 = 2 * c2 + 1
                cn = jnp.minimum(2 * c2 + 2, _NCHUNK - 1)
                drain(srcA, dstA, semSA, semDA)
                issue(i, c1, srcB, dstB, semSB, semDB)
                process(srcA, dstA)
                drain(srcB, dstB, semSB, semDB)
                issue(i, cn, srcA, dstA, semSA, semDA)
                process(srcB, dstB)
                return carry

            lax.fori_loop(0, _NCHUNK // 2, chunk_pair, 0)
            drain(srcA, dstA, semSA, semDA)
            pltpu.sync_copy(
                acc_loc,
                acc_hbm.at[pl.ds((i * _NW + wid) * _F * _N, _F * _N)])

            @pl.when(wid == 0)
            def _(i=i):
                pltpu.sync_copy(den_loc, den_hbm.at[pl.ds(i * _N, _N)])

    return k(ht_r, s1, s2, edges, zeros)


def _finish_body(num_ref, den_ref, out_ref):
    acc = jnp.zeros((_OUT, _N), jnp.float32)
    for i in range(_HOP):
        acc = acc + _COEF[i] * num_ref[i] / den_ref[i]
    out_ref[...] = jnp.where(acc > 0, acc, jnp.exp(acc) - 1.0).T


def _finish(num, den):
    return pl.pallas_call(
        _finish_body,
        out_shape=jax.ShapeDtypeStruct((_N, _OUT), jnp.float32),
    )(num, den)


def kernel(x, k_edge_list, W, a):
    perm = jnp.concatenate([jnp.arange(0, _OUT, 2), jnp.arange(1, _OUT, 2)])
    Wp = W[:, perm]
    ap = a.reshape(2 * _HOP, _OUT)[:, perm].reshape(1, 2 * _OUT * _HOP)
    hp, s1, s2 = _prep(x, Wp, ap)
    hp_r = hp.reshape(_HOP * _NW * (_F // 2) * _N)
    zeros = jnp.zeros((_F * _N,), jnp.float32)
    acc, den = _sc_edges(hp_r, s1.reshape(_HOP * _N), s2.reshape(_HOP * _N),
                         k_edge_list.reshape(_HOP * 2 * _E), zeros)
    num = acc.reshape(_HOP, _OUT, _N)
    return _finish(num, den.reshape(_HOP, 1, _N))


# speedup vs baseline: 10.4311x; 1.0618x over previous
import functools

import jax
import jax.numpy as jnp
from jax import lax
from jax.experimental import pallas as pl
from jax.experimental.pallas import tpu as pltpu
from jax.experimental.pallas import tpu_sc as plsc

_N = 10000
_IN = 128
_OUT = 128
_HOP = 3
_E = 320000
_ALPHA = 0.2
_NC = 2
_NS = 16
_NW = _NC * _NS
_F = _OUT // _NW
_CH = 3200
_NCHUNK = _E // _CH
_GPI = 8
_COEF = (0.5, 0.25, 0.125)


def _prep_body(x_ref, w_ref, a_ref, el_ref, hp_ref, s1_ref, s2_ref, ep_ref):
    xb = x_ref[...]
    _H = _OUT // 2
    for i in range(_HOP):
        ep_ref[i] = el_ref[i, 0] | (el_ref[i, 1] << 16)
        wi_e = w_ref[i * _IN:(i + 1) * _IN, :_H]
        wi_o = w_ref[i * _IN:(i + 1) * _IN, _H:]
        hte = lax.dot_general(wi_e, xb, (((0,), (1,)), ((), ())),
                              preferred_element_type=jnp.float32)
        hto = lax.dot_general(wi_o, xb, (((0,), (1,)), ((), ())),
                              preferred_element_type=jnp.float32)
        hue = lax.bitcast_convert_type(
            hte.astype(jnp.bfloat16), jnp.uint16).astype(jnp.uint32)
        huo = lax.bitcast_convert_type(
            hto.astype(jnp.bfloat16), jnp.uint16).astype(jnp.uint32)
        hp_ref[i] = lax.bitcast_convert_type(hue | (huo << 16), jnp.int32)
        a1e = a_ref[:, 2 * i * _OUT:2 * i * _OUT + _H]
        a1o = a_ref[:, 2 * i * _OUT + _H:2 * i * _OUT + _OUT]
        a2e = a_ref[:, 2 * i * _OUT + _OUT:2 * i * _OUT + _OUT + _H]
        a2o = a_ref[:, 2 * i * _OUT + _OUT + _H:2 * (i + 1) * _OUT]
        s1_ref[i] = (
            lax.dot_general(a1e, hte, (((1,), (0,)), ((), ())),
                            preferred_element_type=jnp.float32)
            + lax.dot_general(a1o, hto, (((1,), (0,)), ((), ())),
                              preferred_element_type=jnp.float32))
        s2_ref[i] = (
            lax.dot_general(a2e, hte, (((1,), (0,)), ((), ())),
                            preferred_element_type=jnp.float32)
            + lax.dot_general(a2o, hto, (((1,), (0,)), ((), ())),
                              preferred_element_type=jnp.float32))


def _prep(x, W, a, el):
    return pl.pallas_call(
        _prep_body,
        out_shape=[
            jax.ShapeDtypeStruct((_HOP, _OUT // 2, _N), jnp.int32),
            jax.ShapeDtypeStruct((_HOP, 1, _N), jnp.float32),
            jax.ShapeDtypeStruct((_HOP, 1, _N), jnp.float32),
            jax.ShapeDtypeStruct((_HOP, _E), jnp.int32),
        ],
    )(x, W, a, el)


def _sc_edges(ht_r, s1, s2, edges, zeros):
    mesh = plsc.VectorSubcoreMesh(core_axis_name="c", subcore_axis_name="s")

    @functools.partial(
        pl.kernel,
        out_type=[
            jax.ShapeDtypeStruct((_HOP * _NW * _F * _N,), jnp.float32),
            jax.ShapeDtypeStruct((_HOP * _N,), jnp.float32),
        ],
        mesh=mesh,
        compiler_params=pltpu.CompilerParams(needs_layout_passes=False),
        scratch_types=[
            pltpu.VMEM(((_F // 2) * _N,), jnp.int32),
            pltpu.VMEM((_F * _N,), jnp.float32),
            pltpu.VMEM((_N,), jnp.float32),
            pltpu.VMEM((_N,), jnp.float32),
            pltpu.VMEM((_N,), jnp.float32),
            pltpu.VMEM((_CH,), jnp.int32),
            pltpu.VMEM((_CH,), jnp.int32),
            pltpu.SemaphoreType.DMA,
            pltpu.SemaphoreType.DMA,
        ],
    )
    def k(ht_hbm, s1_hbm, s2_hbm, e_hbm, z_hbm, acc_hbm, den_hbm,
          h_loc, acc_loc, s1_loc, s2_loc, den_loc,
          epA, epB, semA, semB):
        wid = lax.axis_index("s") * _NC + lax.axis_index("c")

        def issue(i, c, ebuf, sem):
            pltpu.async_copy(
                e_hbm.at[pl.ds(i * _E + c * _CH, _CH)], ebuf, sem)

        def drain(ebuf, sem):
            pltpu.make_async_copy(e_hbm.at[pl.ds(0, _CH)], ebuf, sem).wait()

        def group(eb, base):
            pk = eb[pl.ds(base, 16)]
            src_v = pk & 0xFFFF
            dst_v = jnp.right_shift(pk, 16)
            s1v = plsc.load_gather(s1_loc, [src_v])
            s2v = plsc.load_gather(s2_loc, [dst_v])
            sc = s1v + s2v
            w = jnp.exp(-jnp.maximum(sc, _ALPHA * sc))
            plsc.addupdate_scatter(den_loc, [src_v], w)
            for p in range(_F // 2):
                pw = plsc.load_gather(h_loc.at[pl.ds(p * _N, _N)], [dst_v])
                va, vb = plsc.unpack(plsc.bitcast(pw, jnp.bfloat16),
                                     format=plsc.PackFormat.INTERLEAVED,
                                     preferred_element_type=jnp.float32)
                plsc.addupdate_scatter(acc_loc.at[pl.ds(2 * p * _N, _N)],
                                       [src_v], va * w)
                plsc.addupdate_scatter(acc_loc.at[pl.ds((2 * p + 1) * _N, _N)],
                                       [src_v], vb * w)

        def process(eb):
            @plsc.parallel_loop(0, _CH // 16, 1, unroll=_GPI)
            def _body(g):
                group(eb, g * 16)

        _FP = _F // 2
        for i in range(_HOP):
            pltpu.sync_copy(
                ht_hbm.at[pl.ds((i * _NW + wid) * _FP * _N, _FP * _N)], h_loc)
            pltpu.sync_copy(s1_hbm.at[pl.ds(i * _N, _N)], s1_loc)
            pltpu.sync_copy(s2_hbm.at[pl.ds(i * _N, _N)], s2_loc)
            pltpu.sync_copy(z_hbm, acc_loc)
            pltpu.sync_copy(z_hbm.at[pl.ds(0, _N)], den_loc)

            issue(i, 0, epA, semA)

            def chunk_pair(c2, carry, i=i):
                c1 = 2 * c2 + 1
                cn = jnp.minimum(2 * c2 + 2, _NCHUNK - 1)
                drain(epA, semA)
                issue(i, c1, epB, semB)
                process(epA)
                drain(epB, semB)
                issue(i, cn, epA, semA)
                process(epB)
                return carry

            lax.fori_loop(0, _NCHUNK // 2, chunk_pair, 0)
            drain(epA, semA)
            pltpu.sync_copy(
                acc_loc,
                acc_hbm.at[pl.ds((i * _NW + wid) * _F * _N, _F * _N)])

            @pl.when(wid == 0)
            def _(i=i):
                pltpu.sync_copy(den_loc, den_hbm.at[pl.ds(i * _N, _N)])

    return k(ht_r, s1, s2, edges, zeros)


def _finish_body(num_ref, den_ref, out_ref):
    acc = jnp.zeros((_OUT, _N), jnp.float32)
    for i in range(_HOP):
        acc = acc + _COEF[i] * num_ref[i] / den_ref[i]
    out_ref[...] = jnp.where(acc > 0, acc, jnp.exp(acc) - 1.0).T


def _finish(num, den):
    return pl.pallas_call(
        _finish_body,
        out_shape=jax.ShapeDtypeStruct((_N, _OUT), jnp.float32),
    )(num, den)


def kernel(x, k_edge_list, W, a):
    perm = jnp.concatenate([jnp.arange(0, _OUT, 2), jnp.arange(1, _OUT, 2)])
    Wp = W[:, perm]
    ap = a.reshape(2 * _HOP, _OUT)[:, perm].reshape(1, 2 * _OUT * _HOP)
    hp, s1, s2, ep = _prep(x, Wp, ap, k_edge_list)
    hp_r = hp.reshape(_HOP * _NW * (_F // 2) * _N)
    zeros = jnp.zeros((_F * _N,), jnp.float32)
    acc, den = _sc_edges(hp_r, s1.reshape(_HOP * _N), s2.reshape(_HOP * _N),
                         ep.reshape(_HOP * _E), zeros)
    num = acc.reshape(_HOP, _OUT, _N)
    return _finish(num, den.reshape(_HOP, 1, _N))


# speedup vs baseline: 11.7872x; 1.1300x over previous
import functools

import jax
import jax.numpy as jnp
from jax import lax
from jax.experimental import pallas as pl
from jax.experimental.pallas import tpu as pltpu
from jax.experimental.pallas import tpu_sc as plsc

_N = 10000
_IN = 128
_OUT = 128
_HOP = 3
_E = 320000
_ALPHA = 0.2
_NC = 2
_NS = 16
_NW = _NC * _NS
_F = _OUT // _NW
_CH = 3200
_NCHUNK = _E // _CH
_GPI = 8
_COEF = (0.5, 0.25, 0.125)


def _prep_body(x_ref, w_ref, a_ref, el_ref, hp_ref, s1_ref, s2_ref, ep_ref):
    xb = x_ref[...]
    _H = _OUT // 2
    for i in range(_HOP):
        ep_ref[i] = el_ref[i, 0] | (el_ref[i, 1] << 16)
        wi_e = w_ref[i * _IN:(i + 1) * _IN, :_H]
        wi_o = w_ref[i * _IN:(i + 1) * _IN, _H:]
        hte = lax.dot_general(wi_e, xb, (((0,), (1,)), ((), ())),
                              preferred_element_type=jnp.float32)
        hto = lax.dot_general(wi_o, xb, (((0,), (1,)), ((), ())),
                              preferred_element_type=jnp.float32)
        hue = lax.bitcast_convert_type(
            hte.astype(jnp.bfloat16), jnp.uint16).astype(jnp.uint32)
        huo = lax.bitcast_convert_type(
            hto.astype(jnp.bfloat16), jnp.uint16).astype(jnp.uint32)
        hp_ref[i] = lax.bitcast_convert_type(hue | (huo << 16), jnp.int32)
        a1e = a_ref[:, 2 * i * _OUT:2 * i * _OUT + _H]
        a1o = a_ref[:, 2 * i * _OUT + _H:2 * i * _OUT + _OUT]
        a2e = a_ref[:, 2 * i * _OUT + _OUT:2 * i * _OUT + _OUT + _H]
        a2o = a_ref[:, 2 * i * _OUT + _OUT + _H:2 * (i + 1) * _OUT]
        s1_ref[i] = (
            lax.dot_general(a1e, hte, (((1,), (0,)), ((), ())),
                            preferred_element_type=jnp.float32)
            + lax.dot_general(a1o, hto, (((1,), (0,)), ((), ())),
                              preferred_element_type=jnp.float32))
        s2_ref[i] = (
            lax.dot_general(a2e, hte, (((1,), (0,)), ((), ())),
                            preferred_element_type=jnp.float32)
            + lax.dot_general(a2o, hto, (((1,), (0,)), ((), ())),
                              preferred_element_type=jnp.float32))


def _prep(x, W, a, el):
    return pl.pallas_call(
        _prep_body,
        out_shape=[
            jax.ShapeDtypeStruct((_HOP, _OUT // 2, _N), jnp.int32),
            jax.ShapeDtypeStruct((_HOP, 1, _N), jnp.float32),
            jax.ShapeDtypeStruct((_HOP, 1, _N), jnp.float32),
            jax.ShapeDtypeStruct((_HOP, _E), jnp.int32),
        ],
    )(x, W, a, el)


def _sc_edges(ht_r, s1, s2, edges, zeros):
    mesh = plsc.VectorSubcoreMesh(core_axis_name="c", subcore_axis_name="s")

    @functools.partial(
        pl.kernel,
        out_type=[
            jax.ShapeDtypeStruct((_HOP * _NW * _F * _N,), jnp.float32),
            jax.ShapeDtypeStruct((_HOP * _NS * _N,), jnp.float32),
        ],
        mesh=mesh,
        compiler_params=pltpu.CompilerParams(needs_layout_passes=False),
        scratch_types=[
            pltpu.VMEM(((_F // 2) * _N,), jnp.int32),
            pltpu.VMEM((_F * _N,), jnp.float32),
            pltpu.VMEM((_N,), jnp.float32),
            pltpu.VMEM((_N,), jnp.float32),
            pltpu.VMEM((_N,), jnp.float32),
            pltpu.VMEM((_CH,), jnp.int32),
            pltpu.VMEM((_CH,), jnp.int32),
            pltpu.VMEM((_CH,), jnp.float32),
            pltpu.VMEM((_CH,), jnp.float32),
            pltpu.VMEM_SHARED((_E,), jnp.float32),
            pltpu.SemaphoreType.DMA,
            pltpu.SemaphoreType.DMA,
            pltpu.SemaphoreType.DMA,
            pltpu.SemaphoreType.DMA,
        ],
    )
    def k(ht_hbm, s1_hbm, s2_hbm, e_hbm, z_hbm, acc_hbm, den_hbm,
          h_loc, acc_loc, s1_loc, s2_loc, den_loc,
          epA, epB, wbA, wbB, w_sp, semA, semB, semWA, semWB):
        cid = lax.axis_index("c")
        sid = lax.axis_index("s")
        wid = sid * _NC + cid
        _P1CH = 2000
        _P1SPAN = _E // _NS

        def issue(i, c, ebuf, wbuf, esem, wsem):
            pltpu.async_copy(
                e_hbm.at[pl.ds(i * _E + c * _CH, _CH)], ebuf, esem)
            pltpu.async_copy(w_sp.at[pl.ds(c * _CH, _CH)], wbuf, wsem)

        def drain(ebuf, wbuf, esem, wsem):
            pltpu.make_async_copy(e_hbm.at[pl.ds(0, _CH)], ebuf, esem).wait()
            pltpu.make_async_copy(w_sp.at[pl.ds(0, _CH)], wbuf, wsem).wait()

        def wgroup(base):
            pk = epA[pl.ds(base, 16)]
            src_v = pk & 0xFFFF
            dst_v = jnp.right_shift(pk, 16)
            s1v = plsc.load_gather(s1_loc, [src_v])
            s2v = plsc.load_gather(s2_loc, [dst_v])
            sc = s1v + s2v
            w = jnp.exp(-jnp.maximum(sc, _ALPHA * sc))
            wbA[pl.ds(base, 16)] = w
            plsc.addupdate_scatter(den_loc, [src_v], w)

        def group(eb, wb, base):
            pk = eb[pl.ds(base, 16)]
            src_v = pk & 0xFFFF
            dst_v = jnp.right_shift(pk, 16)
            w = wb[pl.ds(base, 16)]
            for p in range(_F // 2):
                pw = plsc.load_gather(h_loc.at[pl.ds(p * _N, _N)], [dst_v])
                va, vb = plsc.unpack(plsc.bitcast(pw, jnp.bfloat16),
                                     format=plsc.PackFormat.INTERLEAVED,
                                     preferred_element_type=jnp.float32)
                plsc.addupdate_scatter(acc_loc.at[pl.ds(2 * p * _N, _N)],
                                       [src_v], va * w)
                plsc.addupdate_scatter(acc_loc.at[pl.ds((2 * p + 1) * _N, _N)],
                                       [src_v], vb * w)

        def process(eb, wb):
            @plsc.parallel_loop(0, _CH // 16, 1, unroll=_GPI)
            def _body(g):
                group(eb, wb, g * 16)

        _FP = _F // 2
        for i in range(_HOP):
            pltpu.sync_copy(
                ht_hbm.at[pl.ds((i * _NW + wid) * _FP * _N, _FP * _N)], h_loc)
            pltpu.sync_copy(s1_hbm.at[pl.ds(i * _N, _N)], s1_loc)
            pltpu.sync_copy(s2_hbm.at[pl.ds(i * _N, _N)], s2_loc)
            pltpu.sync_copy(z_hbm, acc_loc)
            pltpu.sync_copy(z_hbm.at[pl.ds(0, _N)], den_loc)

            def wchunk(c, carry, i=i):
                base = sid * _P1SPAN + c * _P1CH
                pltpu.sync_copy(e_hbm.at[pl.ds(i * _E + base, _P1CH)],
                                epA.at[pl.ds(0, _P1CH)])

                @plsc.parallel_loop(0, _P1CH // 16, 1, unroll=5)
                def _wbody(g):
                    wgroup(g * 16)

                pltpu.sync_copy(wbA.at[pl.ds(0, _P1CH)],
                                w_sp.at[pl.ds(base, _P1CH)])
                return carry

            lax.fori_loop(0, _P1SPAN // _P1CH, wchunk, 0)

            @pl.when(cid == 0)
            def _(i=i):
                pltpu.sync_copy(den_loc,
                                den_hbm.at[pl.ds((i * _NS + sid) * _N, _N)])

            plsc.subcore_barrier()

            issue(i, 0, epA, wbA, semA, semWA)

            def chunk_pair(c2, carry, i=i):
                c1 = 2 * c2 + 1
                cn = jnp.minimum(2 * c2 + 2, _NCHUNK - 1)
                drain(epA, wbA, semA, semWA)
                issue(i, c1, epB, wbB, semB, semWB)
                process(epA, wbA)
                drain(epB, wbB, semB, semWB)
                issue(i, cn, epA, wbA, semA, semWA)
                process(epB, wbB)
                return carry

            lax.fori_loop(0, _NCHUNK // 2, chunk_pair, 0)
            drain(epA, wbA, semA, semWA)
            pltpu.sync_copy(
                acc_loc,
                acc_hbm.at[pl.ds((i * _NW + wid) * _F * _N, _F * _N)])
            plsc.subcore_barrier()

    return k(ht_r, s1, s2, edges, zeros)


def _finish_body(num_ref, den_ref, out_ref):
    acc = jnp.zeros((_OUT, _N), jnp.float32)
    for i in range(_HOP):
        den = jnp.sum(den_ref[i], axis=0, keepdims=True)
        acc = acc + _COEF[i] * num_ref[i] / den
    out_ref[...] = jnp.where(acc > 0, acc, jnp.exp(acc) - 1.0).T


def _finish(num, den):
    return pl.pallas_call(
        _finish_body,
        out_shape=jax.ShapeDtypeStruct((_N, _OUT), jnp.float32),
    )(num, den)


def kernel(x, k_edge_list, W, a):
    perm = jnp.concatenate([jnp.arange(0, _OUT, 2), jnp.arange(1, _OUT, 2)])
    Wp = W[:, perm]
    ap = a.reshape(2 * _HOP, _OUT)[:, perm].reshape(1, 2 * _OUT * _HOP)
    hp, s1, s2, ep = _prep(x, Wp, ap, k_edge_list)
    hp_r = hp.reshape(_HOP * _NW * (_F // 2) * _N)
    zeros = jnp.zeros((_F * _N,), jnp.float32)
    acc, den = _sc_edges(hp_r, s1.reshape(_HOP * _N), s2.reshape(_HOP * _N),
                         ep.reshape(_HOP * _E), zeros)
    num = acc.reshape(_HOP, _OUT, _N)
    return _finish(num, den.reshape(_HOP, _NS, _N))
